# baseline (speedup 1.0000x reference)
import jax
import jax.numpy as jnp
import numpy as np
from jax import lax
from jax.experimental import pallas as pl
from jax.experimental.pallas import tpu as pltpu

_BMA = 200
_BM = 400
_BJ = 2000
_BK = 1024
_NKB = 10
_TAIL = 10000 - 9 * _BK
_NBUF = 2
_SENT = _NKB


def _k_start(J):
    return min(_BJ * J // _BK, _NKB - 1)


def _ab_body(ra_ref, zo_ref, jr_ref, kr_ref,
             a_ref, x_ref, w1_ref, b1_ref, w2_ref, adja_ref, adj_ref,
             b2_ref, z_ref,
             cm_ref, mz_ref, buf_ref, sem_ref, tbuf_ref,
             tsem_ref):
    t = pl.program_id(0)
    nsteps = pl.num_programs(0)
    L = mz_ref.shape[-1] // 2
    nsub = _BJ // _BMA

    def _make(tt, slot):
        return pltpu.make_async_copy(
            adj_ref.at[pl.ds(pl.multiple_of(jr_ref[tt] * _BJ, _BJ), _BJ),
                       pl.ds(pl.multiple_of(kr_ref[tt] * _BK, _BK), _BK)],
            buf_ref.at[slot],
            sem_ref.at[slot],
        )

    def _make_tail(tt):
        return pltpu.make_async_copy(
            adj_ref.at[pl.ds(pl.multiple_of(jr_ref[tt] * _BJ, _BJ), _BJ),
                       pl.ds(9 * _BK, _TAIL)],
            tbuf_ref,
            tsem_ref,
        )

    def _start(tt):
        tts = jnp.minimum(tt, nsteps - 1)
        in_range = tt < nsteps

        @pl.when(jnp.logical_and(in_range, kr_ref[tts] < _NKB - 1))
        def _():
            _make(tts, tts % _NBUF).start()

        @pl.when(jnp.logical_and(in_range, kr_ref[tts] == _NKB - 1))
        def _():
            _make_tail(tts).start()

    @pl.when(t == 0)
    def _():
        for w in range(_NBUF - 1):
            _start(w)

    _start(t + _NBUF - 1)

    @pl.when(kr_ref[t] == _SENT)
    def _():
        j = ra_ref[t]

        @pl.when(j == 0)
        def _():
            cm_ref[:, L:] = jnp.dot(x_ref[...], w1_ref[...],
                                    preferred_element_type=jnp.float32)
            cm_ref[:, :L] = jnp.zeros_like(cm_ref[:, :L])

        @pl.when(jnp.logical_and(j > 0, j % nsub == 0))
        def _():
            c_prev = _BK * jnp.minimum(_BJ * (j // nsub - 1) // _BK,
                                       _NKB - 1)
            c_cur = _BK * jnp.minimum(_BJ * (j // nsub) // _BK, _NKB - 1)
            adv = c_cur - c_prev

            @pl.when(adv >= _BK)
            def _():
                rows = pl.ds(pl.multiple_of(c_prev, _BK), _BK)
                cm_ref[rows, :L] = mz_ref[rows, :L]

            @pl.when(adv == 2 * _BK)
            def _():
                rows = pl.ds(pl.multiple_of(c_prev + _BK, _BK), _BK)
                cm_ref[rows, :L] = mz_ref[rows, :L]

        hz = jnp.dot(adja_ref[...], cm_ref[...],
                     preferred_element_type=jnp.float32)
        rows = pl.ds(pl.multiple_of(j * _BMA, _BMA), _BMA)
        mz_ref[rows, L:] = hz[:, :L]
        h = hz[:, L:] + b1_ref[...]
        h = jnp.where(h >= 0, h, a_ref[0, 0] * h)
        mz_ref[rows, :L] = jnp.dot(h, w2_ref[...],
                                   preferred_element_type=jnp.float32)

    slot = t % _NBUF
    kk = kr_ref[t]
    jj = jr_ref[t]
    k_first = jnp.minimum(jj * _BJ // _BK, _NKB - 1)
    jrows = pl.ds(pl.multiple_of(jj * _BJ, _BJ), _BJ)

    @pl.when(kk < _NKB - 1)
    def _():
        _make(t, slot).wait()

    @pl.when(kk == _NKB - 1)
    def _():
        _make_tail(t).wait()

    @pl.when(jnp.logical_and(kk == k_first, kk < _SENT))
    def _():
        z_ref[...] = mz_ref[jrows, L:] + b2_ref[...]

    @pl.when(kk < _NKB - 1)
    def _():
        mk = mz_ref[pl.ds(pl.multiple_of(kk * _BK, _BK), _BK), :L]
        z_ref[...] += jnp.dot(buf_ref[slot], mk,
                              preferred_element_type=jnp.float32)

    @pl.when(kk == _NKB - 1)
    def _():
        mk = mz_ref[pl.ds(9 * _BK, _TAIL), :L]
        z_ref[...] += jnp.dot(tbuf_ref[...], mk,
                              preferred_element_type=jnp.float32)


def _sweep_c_body(zj_ref, zall_ref, out_ref):
    out_ref[...] = lax.dot_general(
        zj_ref[...], zall_ref[...],
        (((1,), (1,)), ((), ())),
        preferred_element_type=jnp.float32)


def _schedule(n):
    nba = n // _BMA
    nJ = n // _BJ
    ra, zo, jr, kr = [], [], [], []
    for j in range(nba):
        ra.append(j)
        zo.append(0)
        jr.append(0)
        kr.append(_SENT)
    for J in range(nJ):
        for k in range(_k_start(J), _NKB):
            ra.append(nba - 1)
            zo.append(J)
            jr.append(J)
            kr.append(k)
    return tuple(np.asarray(v, np.int32) for v in (ra, zo, jr, kr))


@jax.jit
def kernel(x, adj, W1, b1, W2, b2, prelu_a):
    N, D = x.shape
    H = W1.shape[1]
    L = W2.shape[1]
    nb = N // _BM

    a2 = prelu_a.reshape(1, 1)
    b1r = b1.reshape(1, H)
    b2r = b2.reshape(1, L)
    ra, zo, jr, kr = _schedule(N)

    z = pl.pallas_call(
        _ab_body,
        grid_spec=pltpu.PrefetchScalarGridSpec(
            num_scalar_prefetch=4,
            grid=(len(ra),),
            in_specs=[
                pl.BlockSpec(memory_space=pltpu.SMEM),
                pl.BlockSpec((N, D), lambda t, r, o, j, k: (0, 0)),
                pl.BlockSpec((D, H), lambda t, r, o, j, k: (0, 0)),
                pl.BlockSpec((1, H), lambda t, r, o, j, k: (0, 0)),
                pl.BlockSpec((H, L), lambda t, r, o, j, k: (0, 0)),
                pl.BlockSpec((_BMA, N), lambda t, r, o, j, k: (r[t], 0)),
                pl.BlockSpec(memory_space=pl.ANY),
                pl.BlockSpec((1, L), lambda t, r, o, j, k: (0, 0)),
            ],
            out_specs=pl.BlockSpec((_BJ, L), lambda t, r, o, j, k: (o[t], 0)),
            scratch_shapes=[
                pltpu.VMEM((N, L + H), jnp.float32),
                pltpu.VMEM((N, 2 * L), jnp.float32),
                pltpu.VMEM((_NBUF, _BJ, _BK), jnp.float32),
                pltpu.SemaphoreType.DMA((_NBUF,)),
                pltpu.VMEM((_BJ, _TAIL), jnp.float32),
                pltpu.SemaphoreType.DMA,
            ],
        ),
        out_shape=jax.ShapeDtypeStruct((N, L), jnp.float32),
    )(ra, zo, jr, kr, a2, x, W1, b1r, W2, adj, adj, b2r)

    adj_hat = pl.pallas_call(
        _sweep_c_body,
        grid=(nb,),
        in_specs=[
            pl.BlockSpec((_BM, L), lambda j: (j, 0)),
            pl.BlockSpec((N, L), lambda j: (0, 0)),
        ],
        out_specs=pl.BlockSpec((_BM, N), lambda j: (j, 0)),
        out_shape=jax.ShapeDtypeStruct((N, N), jnp.float32),
    )(z, z)

    return adj_hat


# speedup vs baseline: 1.0054x; 1.0054x over previous
import jax
import jax.numpy as jnp
import numpy as np
from jax import lax
from jax.experimental import pallas as pl
from jax.experimental.pallas import tpu as pltpu

_BM = 400
_BJ = 2000
_BK = 1024
_NKB = 10
_TAIL = 10000 - 9 * _BK
_NBUF = 3


def _k_start(J):
    return min(_BJ * J // _BK, _NKB - 1)


def _sweep_a_body(a_ref, x_ref, w1_ref, b1_ref, w2_ref, adj_ref,
                  m_ref, zp_ref, cm_ref, mfull_ref):
    j = pl.program_id(0)
    L = m_ref.shape[-1]
    nsub = _BJ // _BM

    @pl.when(j == 0)
    def _():
        cm_ref[:, L:] = jnp.dot(x_ref[...], w1_ref[...],
                                preferred_element_type=jnp.float32)
        cm_ref[:, :L] = jnp.zeros_like(cm_ref[:, :L])

    @pl.when(jnp.logical_and(j > 0, j % nsub == 0))
    def _():
        c_prev = _BK * jnp.minimum(_BJ * (j // nsub - 1) // _BK, _NKB - 1)
        c_cur = _BK * jnp.minimum(_BJ * (j // nsub) // _BK, _NKB - 1)
        adv = c_cur - c_prev

        @pl.when(adv >= _BK)
        def _():
            rows = pl.ds(pl.multiple_of(c_prev, _BK), _BK)
            cm_ref[rows, :L] = mfull_ref[rows, :]

        @pl.when(adv == 2 * _BK)
        def _():
            rows = pl.ds(pl.multiple_of(c_prev + _BK, _BK), _BK)
            cm_ref[rows, :L] = mfull_ref[rows, :]

    hz = jnp.dot(adj_ref[...], cm_ref[...],
                 preferred_element_type=jnp.float32)
    zp_ref[...] = hz[:, :L]
    h = hz[:, L:] + b1_ref[...]
    h = jnp.where(h >= 0, h, a_ref[0, 0] * h)
    mj = jnp.dot(h, w2_ref[...], preferred_element_type=jnp.float32)
    m_ref[...] = mj
    mfull_ref[pl.ds(j * _BM, _BM), :] = mj


def _sweep_b_body(j_ref, k_ref, adj_ref, m_ref, zp_ref, b2_ref,
                  z_ref, buf_ref, sem_ref, tbuf_ref, tsem_ref):
    t = pl.program_id(0)
    nsteps = pl.num_programs(0)

    def _make(tt, slot):
        return pltpu.make_async_copy(
            adj_ref.at[pl.ds(pl.multiple_of(j_ref[tt] * _BJ, _BJ), _BJ),
                       pl.ds(pl.multiple_of(k_ref[tt] * _BK, _BK), _BK)],
            buf_ref.at[slot],
            sem_ref.at[slot],
        )

    def _make_tail(tt):
        return pltpu.make_async_copy(
            adj_ref.at[pl.ds(pl.multiple_of(j_ref[tt] * _BJ, _BJ), _BJ),
                       pl.ds(9 * _BK, _TAIL)],
            tbuf_ref,
            tsem_ref,
        )

    def _start(tt):
        tts = jnp.minimum(tt, nsteps - 1)
        in_range = tt < nsteps

        @pl.when(jnp.logical_and(in_range, k_ref[tts] < _NKB - 1))
        def _():
            _make(tts, tts % _NBUF).start()

        @pl.when(jnp.logical_and(in_range, k_ref[tts] == _NKB - 1))
        def _():
            _make_tail(tts).start()

    @pl.when(t == 0)
    def _():
        for w in range(_NBUF - 1):
            _start(w)

    _start(t + _NBUF - 1)

    slot = t % _NBUF
    kk = k_ref[t]
    jj = j_ref[t]
    k_first = jnp.minimum(jj * _BJ // _BK, _NKB - 1)

    @pl.when(kk < _NKB - 1)
    def _():
        _make(t, slot).wait()

    @pl.when(kk == _NKB - 1)
    def _():
        _make_tail(t).wait()

    @pl.when(kk == k_first)
    def _():
        z_ref[...] = zp_ref[...] + b2_ref[...]

    @pl.when(kk < _NKB - 1)
    def _():
        mk = m_ref[pl.ds(pl.multiple_of(kk * _BK, _BK), _BK), :]
        z_ref[...] += jnp.dot(buf_ref[slot], mk,
                              preferred_element_type=jnp.float32)

    @pl.when(kk == _NKB - 1)
    def _():
        mk = m_ref[pl.ds(9 * _BK, _TAIL), :]
        z_ref[...] += jnp.dot(tbuf_ref[...], mk,
                              preferred_element_type=jnp.float32)


def _sweep_c_body(zj_ref, zall_ref, out_ref):
    out_ref[...] = lax.dot_general(
        zj_ref[...], zall_ref[...],
        (((1,), (1,)), ((), ())),
        preferred_element_type=jnp.float32)


def _pair_list(nJ):
    js, ks = [], []
    for J in range(nJ):
        for k in range(_k_start(J), _NKB):
            js.append(J)
            ks.append(k)
    return np.asarray(js, np.int32), np.asarray(ks, np.int32)


@jax.jit
def kernel(x, adj, W1, b1, W2, b2, prelu_a):
    N, D = x.shape
    H = W1.shape[1]
    L = W2.shape[1]
    nb = N // _BM
    nJ = N // _BJ

    a2 = prelu_a.reshape(1, 1)
    b1r = b1.reshape(1, H)
    b2r = b2.reshape(1, L)
    j_idx, k_idx = _pair_list(nJ)

    m, zp = pl.pallas_call(
        _sweep_a_body,
        grid=(nb,),
        in_specs=[
            pl.BlockSpec(memory_space=pltpu.SMEM),
            pl.BlockSpec((N, D), lambda j: (0, 0)),
            pl.BlockSpec((D, H), lambda j: (0, 0)),
            pl.BlockSpec((1, H), lambda j: (0, 0)),
            pl.BlockSpec((H, L), lambda j: (0, 0)),
            pl.BlockSpec((_BM, N), lambda j: (j, 0)),
        ],
        out_specs=[
            pl.BlockSpec((_BM, L), lambda j: (j, 0)),
            pl.BlockSpec((_BM, L), lambda j: (j, 0)),
        ],
        out_shape=[
            jax.ShapeDtypeStruct((N, L), jnp.float32),
            jax.ShapeDtypeStruct((N, L), jnp.float32),
        ],
        scratch_shapes=[
            pltpu.VMEM((N, L + H), jnp.float32),
            pltpu.VMEM((N, L), jnp.float32),
        ],
    )(a2, x, W1, b1r, W2, adj)

    z = pl.pallas_call(
        _sweep_b_body,
        grid_spec=pltpu.PrefetchScalarGridSpec(
            num_scalar_prefetch=2,
            grid=(len(j_idx),),
            in_specs=[
                pl.BlockSpec(memory_space=pl.ANY),
                pl.BlockSpec((N, L), lambda t, jr, kr: (0, 0)),
                pl.BlockSpec((_BJ, L), lambda t, jr, kr: (jr[t], 0)),
                pl.BlockSpec((1, L), lambda t, jr, kr: (0, 0)),
            ],
            out_specs=pl.BlockSpec((_BJ, L), lambda t, jr, kr: (jr[t], 0)),
            scratch_shapes=[
                pltpu.VMEM((_NBUF, _BJ, _BK), jnp.float32),
                pltpu.SemaphoreType.DMA((_NBUF,)),
                pltpu.VMEM((_BJ, _TAIL), jnp.float32),
                pltpu.SemaphoreType.DMA,
            ],
        ),
        out_shape=jax.ShapeDtypeStruct((N, L), jnp.float32),
    )(j_idx, k_idx, adj, m, zp, b2r)

    adj_hat = pl.pallas_call(
        _sweep_c_body,
        grid=(nb,),
        in_specs=[
            pl.BlockSpec((_BM, L), lambda j: (j, 0)),
            pl.BlockSpec((N, L), lambda j: (0, 0)),
        ],
        out_specs=pl.BlockSpec((_BM, N), lambda j: (j, 0)),
        out_shape=jax.ShapeDtypeStruct((N, N), jnp.float32),
    )(z, z)

    return adj_hat
